# Initial kernel scaffold; baseline (speedup 1.0000x reference)
import jax
import jax.numpy as jnp
from jax.experimental import pallas as pl


def kernel(user_ids, history_item_ids, history_mask, pos_item_ids, neg_item_ids, user_emb, item_emb, gender_emb, age_emb, occ_emb, W1, b1, W2, b2, gender_ids, age_ids, occupation_ids):
    raise NotImplementedError("write your pallas kernel here")


# speedup vs baseline: 1.9845x; 1.9845x over previous
import functools

import jax
import jax.numpy as jnp
from jax import lax
from jax.experimental import pallas as pl
from jax.experimental.pallas import tpu as pltpu
from jax.experimental.pallas import tpu_sc as plsc

NUM_USERS = 128
NUM_GENDER = 3
NUM_AGE = 7
NUM_OCC = 21
EMB = 64
HID = 128
FEAT = 16
B = 4096
H = 50
H_PAD = 56

NUM_CORES = 2
NUM_SUBCORES = 16
NW = NUM_CORES * NUM_SUBCORES
ROWS_PER_W = B // NW


def _sc_body(idx_hbm, mask_hbm, pos_hbm, neg_hbm, item_hbm,
             hist_out, pos_out, neg_out,
             idx_v, mask_v, rows0, rows1, acc_v, pn_idx, pn_rows,
             sem0, sem1, sem2):
    cid = lax.axis_index("c")
    sid = lax.axis_index("s")
    wid = sid * NUM_CORES + cid
    base = wid * ROWS_PER_W

    pltpu.sync_copy(idx_hbm.at[pl.ds(base, ROWS_PER_W)], idx_v)
    pltpu.sync_copy(mask_hbm.at[pl.ds(base, ROWS_PER_W)], mask_v)

    def gstart(r, buf, sem):
        pltpu.make_async_copy(item_hbm.at[idx_v.at[r]], buf, sem).start()

    def gwait(buf, sem):
        pltpu.make_async_copy(item_hbm.at[idx_v.at[0]], buf, sem).wait()

    gstart(0, rows0, sem0)
    gstart(1, rows1, sem1)

    pltpu.sync_copy(pos_hbm.at[pl.ds(base, ROWS_PER_W)], pn_idx)
    pltpu.async_copy(item_hbm.at[pn_idx], pn_rows, sem2).wait()
    pltpu.sync_copy(pn_rows, pos_out.at[pl.ds(base, ROWS_PER_W)])
    pltpu.sync_copy(neg_hbm.at[pl.ds(base, ROWS_PER_W)], pn_idx)
    pltpu.async_copy(item_hbm.at[pn_idx], pn_rows, sem2).wait()
    pltpu.sync_copy(pn_rows, neg_out.at[pl.ds(base, ROWS_PER_W)])

    def compute(r, rows):
        accs = [jnp.zeros((16,), jnp.float32) for _ in range(4)]
        for hb, lo in ((0, 0), (16, 0), (32, 0), (40, 8)):
            mvec = mask_v[r, pl.ds(hb, 16)]
            for h2 in range(lo, 16):
                h = hb + h2
                wv = jnp.full((16,), mvec[h2], jnp.float32)
                for j in range(4):
                    accs[j] = accs[j] + rows[h, pl.ds(16 * j, 16)] * wv
        for j, a in enumerate(accs):
            acc_v[r, pl.ds(16 * j, 16)] = a

    def loop_body(i, _):
        r = 2 * i
        gwait(rows0, sem0)
        compute(r, rows0)

        @pl.when(r + 2 < ROWS_PER_W)
        def _s0():
            gstart(r + 2, rows0, sem0)

        gwait(rows1, sem1)
        compute(r + 1, rows1)

        @pl.when(r + 3 < ROWS_PER_W)
        def _s1():
            gstart(r + 3, rows1, sem1)
        return ()

    lax.fori_loop(0, ROWS_PER_W // 2, loop_body, ())

    pltpu.sync_copy(acc_v, hist_out.at[pl.ds(base, ROWS_PER_W)])


def _sc_pool(idx_pad, mask_pad, pos_ids, neg_ids, item_emb):
    f32 = jnp.float32
    mesh = plsc.VectorSubcoreMesh(core_axis_name="c", subcore_axis_name="s")
    kern = functools.partial(
        pl.kernel, mesh=mesh,
        compiler_params=pltpu.CompilerParams(use_tc_tiling_on_sc=False),
        out_type=[jax.ShapeDtypeStruct((B, EMB), f32) for _ in range(3)],
        scratch_types=[
            pltpu.VMEM((ROWS_PER_W, H_PAD), jnp.int32),
            pltpu.VMEM((ROWS_PER_W, H_PAD), f32),
            pltpu.VMEM((H_PAD, EMB), f32),
            pltpu.VMEM((H_PAD, EMB), f32),
            pltpu.VMEM((ROWS_PER_W, EMB), f32),
            pltpu.VMEM((ROWS_PER_W,), jnp.int32),
            pltpu.VMEM((ROWS_PER_W, EMB), f32),
            pltpu.SemaphoreType.DMA,
            pltpu.SemaphoreType.DMA,
            pltpu.SemaphoreType.DMA,
        ],
    )(_sc_body)
    return kern(idx_pad, mask_pad, pos_ids, neg_ids, item_emb)


def _tc_body(uid_ref, mask_ref, hsum_ref, pos_ref, neg_ref,
             uemb_ref, gemb_ref, aemb_ref, oemb_ref,
             gid_ref, aid_ref, oid_ref,
             W1_ref, b1_ref, W2_ref, b2_ref,
             pos_out, neg_out):
    f32 = jnp.float32

    def dot(a, b):
        return jnp.dot(a, b, preferred_element_type=f32)

    def onehot(ids, n):
        return (ids[:, None] == lax.broadcasted_iota(
            jnp.int32, (ids.shape[0], n), 1)).astype(f32)

    g = dot(onehot(gid_ref[:], NUM_GENDER), gemb_ref[:])
    a = dot(onehot(aid_ref[:], NUM_AGE), aemb_ref[:])
    o = dot(onehot(oid_ref[:], NUM_OCC), oemb_ref[:])
    T = (dot(uemb_ref[:], W1_ref[0:EMB, :])
         + dot(g, W1_ref[EMB * 2:EMB * 2 + FEAT, :])
         + dot(a, W1_ref[EMB * 2 + FEAT:EMB * 2 + 2 * FEAT, :])
         + dot(o, W1_ref[EMB * 2 + 2 * FEAT:EMB * 2 + 3 * FEAT, :])
         + b1_ref[:][None, :])

    u1h = onehot(uid_ref[:], NUM_USERS)
    hist_len = jnp.maximum(
        jnp.sum(mask_ref[:], axis=1, keepdims=True), 1.0)
    hist_mean = hsum_ref[:] / hist_len
    pre = dot(u1h, T) + dot(hist_mean, W1_ref[EMB:EMB * 2, :])
    hidden = jnp.maximum(pre, 0.0)
    user_vec = dot(hidden, W2_ref[:]) + b2_ref[:][None, :]
    pos_out[:] = jnp.sum(user_vec * pos_ref[:], axis=1, keepdims=True)
    neg_out[:] = jnp.sum(user_vec * neg_ref[:], axis=1, keepdims=True)


def kernel(user_ids, history_item_ids, history_mask, pos_item_ids,
           neg_item_ids, user_emb, item_emb, gender_emb, age_emb, occ_emb,
           W1, b1, W2, b2, gender_ids, age_ids, occupation_ids):
    f32 = jnp.float32
    pad = H_PAD - H
    idx_pad = jnp.pad(history_item_ids, ((0, 0), (0, pad)))
    mask_pad = jnp.pad(history_mask, ((0, 0), (0, pad)))

    hist_sum, pos_rows, neg_rows = _sc_pool(
        idx_pad, mask_pad, pos_item_ids, neg_item_ids, item_emb)

    pos_s, neg_s = pl.pallas_call(
        _tc_body,
        out_shape=[jax.ShapeDtypeStruct((B, 1), f32) for _ in range(2)],
    )(user_ids, history_mask, hist_sum, pos_rows, neg_rows,
      user_emb, gender_emb, age_emb, occ_emb,
      gender_ids, age_ids, occupation_ids, W1, b1, W2, b2)
    return pos_s.reshape(B), neg_s.reshape(B)
